# baseline (speedup 1.0000x reference)
import functools

import jax
import jax.numpy as jnp
from jax import lax
from jax.experimental import pallas as pl
from jax.experimental.pallas import tpu as pltpu
from jax.experimental.pallas import tpu_sc as plsc

B = 4
N = 4096
C = 2048
NW = 32
CPW = 128
LG = CPW // 16
UNITS = B * (C // CPW)
UPW = UNITS // NW
RCHUNK = 128
NCHUNK = N // RCHUNK
RUNROLL = 4


def _sc_cumsum(x2):
    mesh = plsc.VectorSubcoreMesh(core_axis_name="c", subcore_axis_name="s")

    @functools.partial(
        pl.kernel,
        mesh=mesh,
        out_type=jax.ShapeDtypeStruct((B * N, C), jnp.float32),
        scratch_types=[
            pltpu.VMEM((RCHUNK, CPW), jnp.float32),
            pltpu.VMEM((RCHUNK, CPW), jnp.float32),
            pltpu.VMEM((RCHUNK, CPW), jnp.float32),
            pltpu.VMEM((RCHUNK, CPW), jnp.float32),
            pltpu.SemaphoreType.DMA,
            pltpu.SemaphoreType.DMA,
            pltpu.SemaphoreType.DMA,
            pltpu.SemaphoreType.DMA,
        ],
    )
    def k(x_hbm, out_hbm, in0, in1, out0, out1, si0, si1, so0, so1):
        wid = lax.axis_index("s") * 2 + lax.axis_index("c")
        ins, outs, sis, sos = (in0, in1), (out0, out1), (si0, si1), (so0, so1)

        def src(unit, ch):
            b = unit // (C // CPW)
            c0 = pl.multiple_of((unit % (C // CPW)) * CPW, CPW)
            r0 = pl.multiple_of(b * N + ch * RCHUNK, RCHUNK)
            return pl.ds(r0, RCHUNK), pl.ds(c0, CPW)

        T = UPW * NCHUNK
        in_cp = [None, None]
        out_cp = [None, None]
        unit0 = wid * UPW
        in_cp[0] = pltpu.async_copy(x_hbm.at[src(unit0, 0)], ins[0], sis[0])
        accs = None
        for t in range(T):
            u, ch = divmod(t, NCHUNK)
            unit = unit0 + u
            slot = t % 2
            if t + 1 < T:
                nu, nch = divmod(t + 1, NCHUNK)
                nslot = (t + 1) % 2
                in_cp[nslot] = pltpu.async_copy(
                    x_hbm.at[src(unit0 + nu, nch)], ins[nslot], sis[nslot]
                )
            in_cp[slot].wait()
            if out_cp[slot] is not None:
                out_cp[slot].wait()
            if ch == 0:
                accs = tuple(jnp.zeros((16,), jnp.float32) for _ in range(LG))
            bi, bo = ins[slot], outs[slot]

            def body(i, accs, bi=bi, bo=bo):
                r0i = pl.multiple_of(i * RUNROLL, RUNROLL)
                for rr in range(RUNROLL):
                    r = r0i + rr
                    new = []
                    for g in range(LG):
                        v = bi[r, pl.ds(g * 16, 16)]
                        a = accs[g] + v
                        bo[r, pl.ds(g * 16, 16)] = a
                        new.append(a)
                    accs = tuple(new)
                return accs

            accs = lax.fori_loop(0, RCHUNK // RUNROLL, body, accs)
            out_cp[slot] = pltpu.async_copy(bo, out_hbm.at[src(unit, ch)], sos[slot])
        out_cp[0].wait()
        out_cp[1].wait()

    return k(x2)


def kernel(x):
    orig_dtype = x.dtype
    x2 = x.astype(jnp.float32).reshape(B * N, C)
    out = _sc_cumsum(x2)
    return out.reshape(B, N, C).astype(orig_dtype)


# speedup vs baseline: 1.1005x; 1.1005x over previous
import functools

import jax
import jax.numpy as jnp
from jax import lax
from jax.experimental import pallas as pl
from jax.experimental.pallas import tpu as pltpu
from jax.experimental.pallas import tpu_sc as plsc

B = 4
N = 4096
C = 2048
NW = 32
CPW = 128
LG = CPW // 16
CB = C // CPW
UNITS = B * CB
UPW = UNITS // NW
RCHUNK = 128
NCHUNK = N // RCHUNK
T = UPW * NCHUNK


def _sc_cumsum(x2):
    mesh = plsc.VectorSubcoreMesh(core_axis_name="c", subcore_axis_name="s")

    @functools.partial(
        pl.kernel,
        mesh=mesh,
        out_type=jax.ShapeDtypeStruct((B * N, C), jnp.float32),
        scratch_types=[
            pltpu.VMEM((RCHUNK, CPW), jnp.float32),
            pltpu.VMEM((RCHUNK, CPW), jnp.float32),
            pltpu.VMEM((RCHUNK, CPW), jnp.float32),
            pltpu.VMEM((RCHUNK, CPW), jnp.float32),
            pltpu.SemaphoreType.DMA,
            pltpu.SemaphoreType.DMA,
            pltpu.SemaphoreType.DMA,
            pltpu.SemaphoreType.DMA,
        ],
    )
    def k(x_hbm, out_hbm, in0, in1, out0, out1, si0, si1, so0, so1):
        wid = lax.axis_index("s") * 2 + lax.axis_index("c")
        ins, outs, sis, sos = (in0, in1), (out0, out1), (si0, si1), (so0, so1)

        def src(t):
            u = t // NCHUNK
            unit = wid * UPW + u
            b = unit // CB
            c0 = pl.multiple_of((unit % CB) * CPW, CPW)
            r0 = pl.multiple_of(b * N + (t % NCHUNK) * RCHUNK, RCHUNK)
            return (pl.ds(r0, RCHUNK), pl.ds(c0, CPW))

        pltpu.async_copy(x_hbm.at[src(0)], ins[0], sis[0])
        pltpu.async_copy(x_hbm.at[src(1)], ins[1], sis[1])

        def chunk(t, accs, slot):
            bi, bo = ins[slot], outs[slot]
            pltpu.make_async_copy(x_hbm.at[src(t)], bi, sis[slot]).wait()
            @pl.when(t >= 2)
            def _():
                pltpu.make_async_copy(bo, out_hbm.at[src(t)], sos[slot]).wait()

            fresh = (t % NCHUNK) == 0
            accs = tuple(jnp.where(fresh, jnp.zeros((16,), jnp.float32), a)
                         for a in accs)

            def body(r, accs):
                new = []
                for g in range(LG):
                    v = bi[r, pl.ds(g * 16, 16)]
                    a = accs[g] + v
                    bo[r, pl.ds(g * 16, 16)] = a
                    new.append(a)
                return tuple(new)

            accs = lax.fori_loop(0, RCHUNK, body, accs)
            pltpu.async_copy(bo, out_hbm.at[src(t)], sos[slot])

            @pl.when(t + 2 < T)
            def _():
                pltpu.async_copy(x_hbm.at[src(t + 2)], bi, sis[slot])

            return accs

        def pair(j, accs):
            accs = chunk(2 * j, accs, 0)
            accs = chunk(2 * j + 1, accs, 1)
            return accs

        accs0 = tuple(jnp.zeros((16,), jnp.float32) for _ in range(LG))
        lax.fori_loop(0, T // 2, pair, accs0)

        pltpu.make_async_copy(outs[0], out_hbm.at[src(jnp.int32(T - 2))], sos[0]).wait()
        pltpu.make_async_copy(outs[1], out_hbm.at[src(jnp.int32(T - 1))], sos[1]).wait()

    return k(x2)


def kernel(x):
    orig_dtype = x.dtype
    x2 = x.astype(jnp.float32).reshape(B * N, C)
    out = _sc_cumsum(x2)
    return out.reshape(B, N, C).astype(orig_dtype)


# speedup vs baseline: 1.1280x; 1.0250x over previous
import functools

import jax
import jax.numpy as jnp
from jax import lax
from jax.experimental import pallas as pl
from jax.experimental.pallas import tpu as pltpu
from jax.experimental.pallas import tpu_sc as plsc

B = 4
N = 4096
C = 2048
NW = 32
CPW = 256
LG = CPW // 16
CB = C // CPW
UNITS = B * CB
UPW = UNITS // NW
RCHUNK = 64
NCHUNK = N // RCHUNK
T = UPW * NCHUNK


def _sc_cumsum(x2):
    mesh = plsc.VectorSubcoreMesh(core_axis_name="c", subcore_axis_name="s")

    @functools.partial(
        pl.kernel,
        mesh=mesh,
        out_type=jax.ShapeDtypeStruct((B * N, C), jnp.float32),
        scratch_types=[
            pltpu.VMEM((RCHUNK, CPW), jnp.float32),
            pltpu.VMEM((RCHUNK, CPW), jnp.float32),
            pltpu.VMEM((RCHUNK, CPW), jnp.float32),
            pltpu.VMEM((RCHUNK, CPW), jnp.float32),
            pltpu.SemaphoreType.DMA,
            pltpu.SemaphoreType.DMA,
            pltpu.SemaphoreType.DMA,
            pltpu.SemaphoreType.DMA,
        ],
    )
    def k(x_hbm, out_hbm, in0, in1, out0, out1, si0, si1, so0, so1):
        wid = lax.axis_index("s") * 2 + lax.axis_index("c")
        ins, outs, sis, sos = (in0, in1), (out0, out1), (si0, si1), (so0, so1)

        def src(t):
            u = t // NCHUNK
            unit = wid * UPW + u
            b = unit // CB
            c0 = pl.multiple_of((unit % CB) * CPW, CPW)
            r0 = pl.multiple_of(b * N + (t % NCHUNK) * RCHUNK, RCHUNK)
            return (pl.ds(r0, RCHUNK), pl.ds(c0, CPW))

        pltpu.async_copy(x_hbm.at[src(0)], ins[0], sis[0])
        pltpu.async_copy(x_hbm.at[src(1)], ins[1], sis[1])

        def chunk(t, accs, slot):
            bi, bo = ins[slot], outs[slot]
            pltpu.make_async_copy(x_hbm.at[src(t)], bi, sis[slot]).wait()
            @pl.when(t >= 2)
            def _():
                pltpu.make_async_copy(bo, out_hbm.at[src(t)], sos[slot]).wait()

            fresh = (t % NCHUNK) == 0
            accs = tuple(jnp.where(fresh, jnp.zeros((16,), jnp.float32), a)
                         for a in accs)

            def body(r, accs):
                new = []
                for g in range(LG):
                    v = bi[r, pl.ds(g * 16, 16)]
                    a = accs[g] + v
                    bo[r, pl.ds(g * 16, 16)] = a
                    new.append(a)
                return tuple(new)

            accs = lax.fori_loop(0, RCHUNK, body, accs)
            pltpu.async_copy(bo, out_hbm.at[src(t)], sos[slot])

            @pl.when(t + 2 < T)
            def _():
                pltpu.async_copy(x_hbm.at[src(t + 2)], bi, sis[slot])

            return accs

        def pair(j, accs):
            accs = chunk(2 * j, accs, 0)
            accs = chunk(2 * j + 1, accs, 1)
            return accs

        accs0 = tuple(jnp.zeros((16,), jnp.float32) for _ in range(LG))
        lax.fori_loop(0, T // 2, pair, accs0)

        pltpu.make_async_copy(outs[0], out_hbm.at[src(jnp.int32(T - 2))], sos[0]).wait()
        pltpu.make_async_copy(outs[1], out_hbm.at[src(jnp.int32(T - 1))], sos[1]).wait()

    return k(x2)


def kernel(x):
    orig_dtype = x.dtype
    x2 = x.astype(jnp.float32).reshape(B * N, C)
    out = _sc_cumsum(x2)
    return out.reshape(B, N, C).astype(orig_dtype)
